# baseline (speedup 1.0000x reference)
import functools

import jax
import jax.numpy as jnp
from jax import lax
from jax.experimental import pallas as pl
from jax.experimental.pallas import tpu as pltpu

N = 10000
E = 320000
D = 128
DE = 16
L = 3
NEG = 0.2
BS = 10


def _mm_kernel(a_ref, b_ref, o_ref):
    o_ref[...] = jnp.dot(a_ref[...], b_ref[...],
                         preferred_element_type=jnp.float32)


def _matmul(a, b, block_rows=1000):
    m, k = a.shape
    k2, n = b.shape
    grid = (m // block_rows,)
    return pl.pallas_call(
        _mm_kernel,
        grid=grid,
        in_specs=[
            pl.BlockSpec((block_rows, k), lambda i: (i, 0)),
            pl.BlockSpec((k, n), lambda i: (0, 0)),
        ],
        out_specs=pl.BlockSpec((block_rows, n), lambda i: (i, 0)),
        out_shape=jax.ShapeDtypeStruct((m, n), jnp.float32),
    )(a, b)


def _bn(v, g, b):
    m = jnp.mean(v, axis=0)
    va = jnp.var(v, axis=0)
    return g * (v - m) / jnp.sqrt(va + 1e-5) + b


def kernel(x, edge_index, edge_attr, batch, fc_node_W, fc_node_b, fc_edge_W,
           fc_edge_b, bn_node_g, bn_node_b, bn_edge_g, bn_edge_b, gat_W,
           gat_a_src, gat_a_dst, gat_a_edge, gat_b):
    src = edge_index[0]
    dst = edge_index[1]
    h = _bn(_matmul(x, fc_node_W) + fc_node_b, bn_node_g, bn_node_b)
    e = _bn(edge_attr @ fc_edge_W + fc_edge_b, bn_edge_g, bn_edge_b)
    for l in range(L):
        hw = _matmul(h, gat_W[l])
        s = hw @ gat_a_src[l]
        dvec = hw @ gat_a_dst[l]
        ea = e @ gat_a_edge[l]
        logits = s[src] + dvec[dst] + ea
        logits = jax.nn.leaky_relu(logits, NEG)
        mx = jax.ops.segment_max(logits, dst, num_segments=N)
        ex = jnp.exp(logits - mx[dst])
        den = jax.ops.segment_sum(ex, dst, num_segments=N)
        alpha = ex / (den[dst] + 1e-16)
        out = jax.ops.segment_sum(alpha[:, None] * hw[src], dst,
                                  num_segments=N)
        h = h + out + gat_b[l]
    return h.reshape(BS, -1, D)


# speedup vs baseline: 16.7458x; 16.7458x over previous
import functools

import jax
import jax.numpy as jnp
from jax import lax
from jax.experimental import pallas as pl
from jax.experimental.pallas import tpu as pltpu
from jax.experimental.pallas import tpu_sc as plsc

N = 10000
E = 320000
D = 128
DE = 16
L = 3
NEG = 0.2
BS = 10

NC = 2
NS = 16
NW = NC * NS
EP = E // NW
B = 80
NBLK = EP // B
NP = 10240
NPT = NP // NS
VEC = 16

_mesh = plsc.VectorSubcoreMesh(core_axis_name="c", subcore_axis_name="s",
                               num_cores=NC, num_subcores=NS)
_sc_params = pltpu.CompilerParams(needs_layout_passes=False)


def _wid():
    return lax.axis_index("s") * NC + lax.axis_index("c")


def _stats_kernel(x_ref, xtx_ref, cs_ref):
    @pl.when(pl.program_id(0) == 0)
    def _init():
        xtx_ref[...] = jnp.zeros_like(xtx_ref)
        cs_ref[...] = jnp.zeros_like(cs_ref)

    xb = x_ref[...]
    xtx_ref[...] += lax.dot_general(xb, xb, (((0,), (0,)), ((), ())),
                                    preferred_element_type=jnp.float32)
    cs = jnp.sum(xb, axis=0, keepdims=True)
    cs_ref[...] += jnp.broadcast_to(cs, cs_ref.shape)


def _stats(x, blk):
    m, k = x.shape
    return pl.pallas_call(
        _stats_kernel,
        grid=(m // blk,),
        in_specs=[pl.BlockSpec((blk, k), lambda i: (i, 0))],
        out_specs=[pl.BlockSpec((k, k), lambda i: (0, 0)),
                   pl.BlockSpec((8, k), lambda i: (0, 0))],
        out_shape=[jax.ShapeDtypeStruct((k, k), jnp.float32),
                   jax.ShapeDtypeStruct((8, k), jnp.float32)],
    )(x)


def _affine_kernel(x_ref, m_ref, c_ref, o_ref):
    o_ref[...] = jnp.dot(x_ref[...], m_ref[...],
                         preferred_element_type=jnp.float32) + c_ref[0:1, :]


def _affine(x, m, c, blk):
    rows, k = x.shape
    k2, n = m.shape
    cb = jnp.broadcast_to(c[None, :], (8, n))
    return pl.pallas_call(
        _affine_kernel,
        grid=(rows // blk,),
        in_specs=[pl.BlockSpec((blk, k), lambda i: (i, 0)),
                  pl.BlockSpec((k, n), lambda i: (0, 0)),
                  pl.BlockSpec((8, n), lambda i: (0, 0))],
        out_specs=pl.BlockSpec((blk, n), lambda i: (i, 0)),
        out_shape=jax.ShapeDtypeStruct((rows, n), jnp.float32),
    )(x, m, cb)


def _layer_in_kernel(h_ref, w_ref, a2_ref, hw_ref, sd_ref):
    hw = jnp.dot(h_ref[...], w_ref[...], preferred_element_type=jnp.float32)
    hw_ref[...] = hw
    sd_ref[...] = jnp.dot(hw, a2_ref[...], preferred_element_type=jnp.float32)


def _layer_in(h, w, a2, blk=1000):
    return pl.pallas_call(
        _layer_in_kernel,
        grid=(N // blk,),
        in_specs=[pl.BlockSpec((blk, D), lambda i: (i, 0)),
                  pl.BlockSpec((D, D), lambda i: (0, 0)),
                  pl.BlockSpec((D, 8), lambda i: (0, 0))],
        out_specs=[pl.BlockSpec((blk, D), lambda i: (i, 0)),
                   pl.BlockSpec((blk, 8), lambda i: (i, 0))],
        out_shape=[jax.ShapeDtypeStruct((N, D), jnp.float32),
                   jax.ShapeDtypeStruct((N, 8), jnp.float32)],
    )(h, w, a2)


def _upd_kernel(h_ref, o0_ref, o1_ref, b_ref, o_ref):
    o_ref[...] = h_ref[...] + o0_ref[...] + o1_ref[...] + b_ref[0:1, :]


def _update(h, o0, o1, bvec, blk=1000):
    bb = jnp.broadcast_to(bvec[None, :], (8, D))
    return pl.pallas_call(
        _upd_kernel,
        grid=(N // blk,),
        in_specs=[pl.BlockSpec((blk, D), lambda i: (i, 0)),
                  pl.BlockSpec((blk, D), lambda i: (i, 0)),
                  pl.BlockSpec((blk, D), lambda i: (i, 0)),
                  pl.BlockSpec((8, D), lambda i: (0, 0))],
        out_specs=pl.BlockSpec((blk, D), lambda i: (i, 0)),
        out_shape=jax.ShapeDtypeStruct((N, D), jnp.float32),
    )(h, o0, o1, bb)


def _mt_kernel(d0_ref, d1_ref, tm_ref, mt_ref):
    g = jnp.max(tm_ref[...])
    den = d0_ref[...] + d1_ref[...]
    mt = jnp.log(jnp.maximum(den, 1e-38)) + g
    den2 = den * jnp.exp(g - mt)
    mt_ref[...] = mt + jnp.log(den2 + 1e-16)


def _mt(den_p, tmax):
    d2 = den_p.reshape(NC, NP // 128, 128)
    tm = tmax.reshape(4, 128)
    mt = pl.pallas_call(
        _mt_kernel,
        out_shape=jax.ShapeDtypeStruct((NP // 128, 128), jnp.float32),
    )(d2[0], d2[1], tm)
    return mt.reshape(NP)


@functools.partial(
    pl.kernel,
    out_type=(jax.ShapeDtypeStruct((E,), jnp.float32),
              jax.ShapeDtypeStruct((NW * VEC,), jnp.float32)),
    mesh=_mesh,
    compiler_params=_sc_params,
    scratch_types=(pltpu.VMEM((N,), jnp.float32),
                   pltpu.VMEM((N,), jnp.float32),
                   pltpu.VMEM((EP,), jnp.int32),
                   pltpu.VMEM((EP,), jnp.int32),
                   pltpu.VMEM((EP,), jnp.float32),
                   pltpu.VMEM((EP,), jnp.float32),
                   pltpu.VMEM((VEC,), jnp.float32)),
)
def _sc_logits(s_h, d_h, src_h, dst_h, ea_h, lg_h, tmax_h,
               s_v, d_v, src_v, dst_v, ea_v, lg_v, tm_v):
    wid = _wid()
    base = wid * EP
    pltpu.sync_copy(s_h, s_v)
    pltpu.sync_copy(d_h, d_v)
    pltpu.sync_copy(src_h.at[pl.ds(base, EP)], src_v)
    pltpu.sync_copy(dst_h.at[pl.ds(base, EP)], dst_v)
    pltpu.sync_copy(ea_h.at[pl.ds(base, EP)], ea_v)

    def step(i, vmax):
        sl = pl.ds(i * VEC, VEC)
        s16 = plsc.load_gather(s_v, [src_v[sl]])
        d16 = plsc.load_gather(d_v, [dst_v[sl]])
        lg = s16 + d16 + ea_v[sl]
        lg = jnp.where(lg >= 0, lg, lg * NEG)
        lg_v[sl] = lg
        return jnp.maximum(vmax, lg)

    vmax = lax.fori_loop(0, EP // VEC, step,
                         jnp.full((VEC,), -3e38, jnp.float32))
    tm_v[...] = jnp.broadcast_to(jnp.max(vmax), (VEC,))
    pltpu.sync_copy(lg_v, lg_h.at[pl.ds(base, EP)])
    pltpu.sync_copy(tm_v, tmax_h.at[pl.ds(wid * VEC, VEC)])


@functools.partial(
    pl.kernel,
    out_type=jax.ShapeDtypeStruct((NC, NP), jnp.float32),
    mesh=_mesh,
    compiler_params=_sc_params,
    scratch_types=(pltpu.VMEM((EP,), jnp.float32),
                   pltpu.VMEM((EP,), jnp.float32),
                   pltpu.VMEM((NBLK, B), jnp.int32),
                   pltpu.VMEM((NW * VEC,), jnp.float32),
                   pltpu.VMEM((NPT,), jnp.float32),
                   pltpu.VMEM_SHARED((NP,), jnp.float32)),
)
def _sc_den(lg_h, dst3_h, tmax_h, den_h,
            lg_v, ex_v, dst2_v, tm_v, z_v, den_sh):
    cid = lax.axis_index("c")
    sid = lax.axis_index("s")
    wid = _wid()
    base = wid * EP
    pltpu.sync_copy(lg_h.at[pl.ds(base, EP)], lg_v)
    pltpu.sync_copy(dst3_h.at[wid], dst2_v)
    pltpu.sync_copy(tmax_h, tm_v)

    def zstep(i, _):
        z_v[pl.ds(i * VEC, VEC)] = jnp.zeros((VEC,), jnp.float32)
        return 0
    lax.fori_loop(0, NPT // VEC, zstep, 0)
    pltpu.sync_copy(z_v, den_sh.at[pl.ds(sid * NPT, NPT)])

    def mstep(i, m):
        return jnp.maximum(m, tm_v[pl.ds(i * VEC, VEC)])
    g = jnp.max(lax.fori_loop(0, NW, mstep,
                              jnp.full((VEC,), -3e38, jnp.float32)))

    def estep(i, _):
        sl = pl.ds(i * VEC, VEC)
        ex_v[sl] = jnp.exp(lg_v[sl] - g)
        return 0
    lax.fori_loop(0, EP // VEC, estep, 0)

    plsc.subcore_barrier()

    def sstep(j, _):
        pltpu.sync_copy(ex_v.at[pl.ds(j * B, B)],
                        den_sh.at[dst2_v.at[j]], add=True)
        return 0
    lax.fori_loop(0, NBLK, sstep, 0)

    plsc.subcore_barrier()
    pltpu.sync_copy(den_sh.at[pl.ds(sid * NPT, NPT)],
                    den_h.at[cid, pl.ds(sid * NPT, NPT)])


@functools.partial(
    pl.kernel,
    out_type=jax.ShapeDtypeStruct((E,), jnp.float32),
    mesh=_mesh,
    compiler_params=_sc_params,
    scratch_types=(pltpu.VMEM((EP,), jnp.float32),
                   pltpu.VMEM((NBLK, B), jnp.int32),
                   pltpu.VMEM((NP,), jnp.float32)),
)
def _sc_alpha(lg_h, dst3_h, mt_h, al_h, lg_v, dst2_v, mt_v):
    wid = _wid()
    base = wid * EP
    pltpu.sync_copy(lg_h.at[pl.ds(base, EP)], lg_v)
    pltpu.sync_copy(dst3_h.at[wid], dst2_v)
    pltpu.sync_copy(mt_h, mt_v)

    def astep(j, _):
        for k in range(B // VEC):
            sl = pl.ds(j * B + k * VEC, VEC)
            d16 = dst2_v[j, pl.ds(k * VEC, VEC)]
            m16 = plsc.load_gather(mt_v, [d16])
            lg_v[sl] = jnp.exp(lg_v[sl] - m16)
        return 0
    lax.fori_loop(0, NBLK, astep, 0)
    pltpu.sync_copy(lg_v, al_h.at[pl.ds(base, EP)])


@functools.partial(
    pl.kernel,
    out_type=jax.ShapeDtypeStruct((NC, NP, D), jnp.float32),
    mesh=_mesh,
    compiler_params=_sc_params,
    scratch_types=(pltpu.VMEM((EP,), jnp.float32),
                   pltpu.VMEM((NBLK, B), jnp.int32),
                   pltpu.VMEM((B,), jnp.int32),
                   pltpu.VMEM((B,), jnp.int32),
                   pltpu.VMEM((B, D), jnp.float32),
                   pltpu.VMEM_SHARED((NP, D), jnp.float32),
                   pltpu.SemaphoreType.DMA,
                   pltpu.SemaphoreType.DMA),
)
def _sc_out(al_h, pk3_h, hw_h, out_h,
            al_v, pk2_v, sr_v, dr_v, rb, out_sh, gsem, ssem):
    cid = lax.axis_index("c")
    sid = lax.axis_index("s")
    wid = _wid()
    base = wid * EP
    pltpu.sync_copy(al_h.at[pl.ds(base, EP)], al_v)
    pltpu.sync_copy(pk3_h.at[wid], pk2_v)

    def zrow(r, _):
        for c in range(8):
            rb[r, pl.ds(c * VEC, VEC)] = jnp.zeros((VEC,), jnp.float32)
        return 0
    lax.fori_loop(0, B, zrow, 0)
    for k in range(NPT // B):
        pltpu.sync_copy(rb, out_sh.at[pl.ds(sid * NPT + k * B, B)])
    plsc.subcore_barrier()

    def blk(j, _):
        for k in range(B // VEC):
            sl = pl.ds(k * VEC, VEC)
            p16 = pk2_v[j, sl]
            sr_v[sl] = p16 & 16383
            dr_v[sl] = p16 >> 14
        pltpu.async_copy(hw_h.at[sr_v], rb, gsem).wait()

        def row16(t, _):
            a16 = al_v[pl.ds(j * B + t * VEC, VEC)]
            for i in range(VEC):
                a = a16[i]
                r = t * VEC + i
                for c in range(8):
                    sl = pl.ds(c * VEC, VEC)
                    rb[r, sl] = rb[r, sl] * a
            return 0
        lax.fori_loop(0, B // VEC, row16, 0)
        pltpu.async_copy(rb, out_sh.at[dr_v], ssem, add=True).wait()
        return 0
    lax.fori_loop(0, NBLK, blk, 0)

    plsc.subcore_barrier()
    pltpu.sync_copy(out_sh.at[pl.ds(sid * NPT, NPT)],
                    out_h.at[cid, pl.ds(sid * NPT, NPT)])


def kernel(x, edge_index, edge_attr, batch, fc_node_W, fc_node_b, fc_edge_W,
           fc_edge_b, bn_node_g, bn_node_b, bn_edge_g, bn_edge_b, gat_W,
           gat_a_src, gat_a_dst, gat_a_edge, gat_b):
    src = edge_index[0]
    dst = edge_index[1]
    dst3 = dst.reshape(NW, NBLK, B)
    pk3 = (src + dst * 16384).reshape(NW, NBLK, B)

    xtx, xcs = _stats(x, 1000)
    xmean = xcs[0] / N
    mean_v = xmean @ fc_node_W + fc_node_b
    exx = xtx / N
    var_v = jnp.sum((exx @ fc_node_W) * fc_node_W, axis=0) - (mean_v - fc_node_b) ** 2
    scale = bn_node_g / jnp.sqrt(var_v + 1e-5)
    M2 = fc_node_W * scale[None, :]
    c2 = (fc_node_b - mean_v) * scale + bn_node_b

    ete, ecs = _stats(edge_attr, 8000)
    emean = ecs[0] / E
    mean_e = emean @ fc_edge_W + fc_edge_b
    eee = ete / E
    var_e = jnp.sum((eee @ fc_edge_W) * fc_edge_W, axis=0) - (mean_e - fc_edge_b) ** 2
    scale_e = bn_edge_g / jnp.sqrt(var_e + 1e-5)
    A = jnp.zeros((DE, 8), jnp.float32).at[:, :L].set(gat_a_edge.T)
    M3 = (fc_edge_W * scale_e[None, :]) @ A
    c3 = ((fc_edge_b - mean_e) * scale_e + bn_edge_b) @ A
    ea_all = _affine(edge_attr, M3, c3, 8000)

    h = _affine(x, M2, c2, 1000)

    for l in range(L):
        a2 = jnp.zeros((D, 8), jnp.float32)
        a2 = a2.at[:, 0].set(gat_a_src[l]).at[:, 1].set(gat_a_dst[l])
        hw, sd = _layer_in(h, gat_W[l], a2)
        s = sd[:, 0]
        d = sd[:, 1]
        eal = ea_all[:, l]

        logits, tmax = _sc_logits(s, d, src, dst, eal)
        den_p = _sc_den(logits, dst3, tmax)
        mt = _mt(den_p, tmax)
        alpha = _sc_alpha(logits, dst3, mt)
        out_p = _sc_out(alpha, pk3, hw)
        h = _update(h, out_p[0, :N], out_p[1, :N], gat_b[l])

    return h.reshape(BS, -1, D)


# speedup vs baseline: 21.5111x; 1.2846x over previous
import functools

import jax
import jax.numpy as jnp
from jax import lax
from jax.experimental import pallas as pl
from jax.experimental.pallas import tpu as pltpu
from jax.experimental.pallas import tpu_sc as plsc

N = 10000
E = 320000
D = 128
DE = 16
L = 3
NEG = 0.2
BS = 10

NC = 2
NS = 16
NW = NC * NS
EP = E // NW
B = 80
NBLK = EP // B
NP = 10240
NPT = NP // NS
VEC = 16

_mesh = plsc.VectorSubcoreMesh(core_axis_name="c", subcore_axis_name="s",
                               num_cores=NC, num_subcores=NS)
_sc_params = pltpu.CompilerParams(needs_layout_passes=False)


def _wid():
    return lax.axis_index("s") * NC + lax.axis_index("c")


def _stats_kernel(x_ref, xtx_ref, cs_ref):
    @pl.when(pl.program_id(0) == 0)
    def _init():
        xtx_ref[...] = jnp.zeros_like(xtx_ref)
        cs_ref[...] = jnp.zeros_like(cs_ref)

    xb = x_ref[...]
    xtx_ref[...] += lax.dot_general(xb, xb, (((0,), (0,)), ((), ())),
                                    preferred_element_type=jnp.float32)
    cs = jnp.sum(xb, axis=0, keepdims=True)
    cs_ref[...] += jnp.broadcast_to(cs, cs_ref.shape)


def _stats(x, blk):
    m, k = x.shape
    return pl.pallas_call(
        _stats_kernel,
        grid=(m // blk,),
        in_specs=[pl.BlockSpec((blk, k), lambda i: (i, 0))],
        out_specs=[pl.BlockSpec((k, k), lambda i: (0, 0)),
                   pl.BlockSpec((8, k), lambda i: (0, 0))],
        out_shape=[jax.ShapeDtypeStruct((k, k), jnp.float32),
                   jax.ShapeDtypeStruct((8, k), jnp.float32)],
    )(x)


def _affine_kernel(x_ref, m_ref, c_ref, o_ref):
    o_ref[...] = jnp.dot(x_ref[...], m_ref[...],
                         preferred_element_type=jnp.float32) + c_ref[0:1, :]


def _affine(x, m, c, blk):
    rows, k = x.shape
    k2, n = m.shape
    cb = jnp.broadcast_to(c[None, :], (8, n))
    return pl.pallas_call(
        _affine_kernel,
        grid=(rows // blk,),
        in_specs=[pl.BlockSpec((blk, k), lambda i: (i, 0)),
                  pl.BlockSpec((k, n), lambda i: (0, 0)),
                  pl.BlockSpec((8, n), lambda i: (0, 0))],
        out_specs=pl.BlockSpec((blk, n), lambda i: (i, 0)),
        out_shape=jax.ShapeDtypeStruct((rows, n), jnp.float32),
    )(x, m, cb)


def _layer_in_kernel(h_ref, w_ref, a2_ref, hw_ref, sd_ref):
    hw = jnp.dot(h_ref[...], w_ref[...], preferred_element_type=jnp.float32)
    hw_ref[...] = hw
    sd_ref[...] = jnp.dot(hw, a2_ref[...], preferred_element_type=jnp.float32)


def _layer_in(h, w, a2, blk=1000):
    return pl.pallas_call(
        _layer_in_kernel,
        grid=(N // blk,),
        in_specs=[pl.BlockSpec((blk, D), lambda i: (i, 0)),
                  pl.BlockSpec((D, D), lambda i: (0, 0)),
                  pl.BlockSpec((D, 8), lambda i: (0, 0))],
        out_specs=[pl.BlockSpec((blk, D), lambda i: (i, 0)),
                   pl.BlockSpec((blk, 8), lambda i: (i, 0))],
        out_shape=[jax.ShapeDtypeStruct((N, D), jnp.float32),
                   jax.ShapeDtypeStruct((N, 8), jnp.float32)],
    )(h, w, a2)


def _layer_upd_kernel(h_ref, o0_ref, o1_ref, b_ref, w_ref, a2_ref,
                      hn_ref, hw_ref, sd_ref):
    hn = h_ref[...] + o0_ref[...] + o1_ref[...] + b_ref[0:1, :]
    hn_ref[...] = hn
    hw = jnp.dot(hn, w_ref[...], preferred_element_type=jnp.float32)
    hw_ref[...] = hw
    sd_ref[...] = jnp.dot(hw, a2_ref[...], preferred_element_type=jnp.float32)


def _layer_upd(h, o0, o1, bvec, w, a2, blk=1000):
    bb = jnp.broadcast_to(bvec[None, :], (8, D))
    return pl.pallas_call(
        _layer_upd_kernel,
        grid=(N // blk,),
        in_specs=[pl.BlockSpec((blk, D), lambda i: (i, 0)),
                  pl.BlockSpec((blk, D), lambda i: (i, 0)),
                  pl.BlockSpec((blk, D), lambda i: (i, 0)),
                  pl.BlockSpec((8, D), lambda i: (0, 0)),
                  pl.BlockSpec((D, D), lambda i: (0, 0)),
                  pl.BlockSpec((D, 8), lambda i: (0, 0))],
        out_specs=[pl.BlockSpec((blk, D), lambda i: (i, 0)),
                   pl.BlockSpec((blk, D), lambda i: (i, 0)),
                   pl.BlockSpec((blk, 8), lambda i: (i, 0))],
        out_shape=[jax.ShapeDtypeStruct((N, D), jnp.float32),
                   jax.ShapeDtypeStruct((N, D), jnp.float32),
                   jax.ShapeDtypeStruct((N, 8), jnp.float32)],
    )(h, o0, o1, bb, w, a2)


def _upd_kernel(h_ref, o0_ref, o1_ref, b_ref, o_ref):
    o_ref[...] = h_ref[...] + o0_ref[...] + o1_ref[...] + b_ref[0:1, :]


def _update(h, o0, o1, bvec, blk=1000):
    bb = jnp.broadcast_to(bvec[None, :], (8, D))
    return pl.pallas_call(
        _upd_kernel,
        grid=(N // blk,),
        in_specs=[pl.BlockSpec((blk, D), lambda i: (i, 0)),
                  pl.BlockSpec((blk, D), lambda i: (i, 0)),
                  pl.BlockSpec((blk, D), lambda i: (i, 0)),
                  pl.BlockSpec((8, D), lambda i: (0, 0))],
        out_specs=pl.BlockSpec((blk, D), lambda i: (i, 0)),
        out_shape=jax.ShapeDtypeStruct((N, D), jnp.float32),
    )(h, o0, o1, bb)


def _mt_kernel(d0_ref, d1_ref, tm_ref, mt_ref):
    g = jnp.max(tm_ref[...])
    den = d0_ref[...] + d1_ref[...]
    mt = jnp.log(jnp.maximum(den, 1e-38)) + g
    den2 = den * jnp.exp(g - mt)
    mt_ref[...] = mt + jnp.log(den2 + 1e-16)


def _mt(den_p, tmax):
    d2 = den_p.reshape(NC, NP // 128, 128)
    tm = tmax.reshape(4, 128)
    mt = pl.pallas_call(
        _mt_kernel,
        out_shape=jax.ShapeDtypeStruct((NP // 128, 128), jnp.float32),
    )(d2[0], d2[1], tm)
    return mt.reshape(NP)


@functools.partial(
    pl.kernel,
    out_type=(jax.ShapeDtypeStruct((E,), jnp.float32),
              jax.ShapeDtypeStruct((NW * VEC,), jnp.float32)),
    mesh=_mesh,
    compiler_params=_sc_params,
    scratch_types=(pltpu.VMEM((N,), jnp.float32),
                   pltpu.VMEM((N,), jnp.float32),
                   pltpu.VMEM((EP,), jnp.int32),
                   pltpu.VMEM((EP,), jnp.int32),
                   pltpu.VMEM((EP,), jnp.float32),
                   pltpu.VMEM((EP,), jnp.float32),
                   pltpu.VMEM((VEC,), jnp.float32)),
)
def _sc_logits(s_h, d_h, src_h, dst_h, ea_h, lg_h, tmax_h,
               s_v, d_v, src_v, dst_v, ea_v, lg_v, tm_v):
    wid = _wid()
    base = wid * EP
    pltpu.sync_copy(s_h, s_v)
    pltpu.sync_copy(d_h, d_v)
    pltpu.sync_copy(src_h.at[pl.ds(base, EP)], src_v)
    pltpu.sync_copy(dst_h.at[pl.ds(base, EP)], dst_v)
    pltpu.sync_copy(ea_h.at[pl.ds(base, EP)], ea_v)

    def step(i, vmax):
        sl = pl.ds(i * VEC, VEC)
        s16 = plsc.load_gather(s_v, [src_v[sl]])
        d16 = plsc.load_gather(d_v, [dst_v[sl]])
        lg = s16 + d16 + ea_v[sl]
        lg = jnp.where(lg >= 0, lg, lg * NEG)
        lg_v[sl] = lg
        return jnp.maximum(vmax, lg)

    vmax = lax.fori_loop(0, EP // VEC, step,
                         jnp.full((VEC,), -3e38, jnp.float32))
    tm_v[...] = jnp.broadcast_to(jnp.max(vmax), (VEC,))
    pltpu.sync_copy(lg_v, lg_h.at[pl.ds(base, EP)])
    pltpu.sync_copy(tm_v, tmax_h.at[pl.ds(wid * VEC, VEC)])


@functools.partial(
    pl.kernel,
    out_type=jax.ShapeDtypeStruct((NC, NP), jnp.float32),
    mesh=_mesh,
    compiler_params=_sc_params,
    scratch_types=(pltpu.VMEM((EP,), jnp.float32),
                   pltpu.VMEM((EP,), jnp.float32),
                   pltpu.VMEM((NBLK, B), jnp.int32),
                   pltpu.VMEM((NW * VEC,), jnp.float32),
                   pltpu.VMEM((NPT,), jnp.float32),
                   pltpu.VMEM_SHARED((NP,), jnp.float32)),
)
def _sc_den(lg_h, dst3_h, tmax_h, den_h,
            lg_v, ex_v, dst2_v, tm_v, z_v, den_sh):
    cid = lax.axis_index("c")
    sid = lax.axis_index("s")
    wid = _wid()
    base = wid * EP
    pltpu.sync_copy(lg_h.at[pl.ds(base, EP)], lg_v)
    pltpu.sync_copy(dst3_h.at[wid], dst2_v)
    pltpu.sync_copy(tmax_h, tm_v)

    def zstep(i, _):
        z_v[pl.ds(i * VEC, VEC)] = jnp.zeros((VEC,), jnp.float32)
        return 0
    lax.fori_loop(0, NPT // VEC, zstep, 0)
    pltpu.sync_copy(z_v, den_sh.at[pl.ds(sid * NPT, NPT)])

    def mstep(i, m):
        return jnp.maximum(m, tm_v[pl.ds(i * VEC, VEC)])
    g = jnp.max(lax.fori_loop(0, NW, mstep,
                              jnp.full((VEC,), -3e38, jnp.float32)))

    def estep(i, _):
        sl = pl.ds(i * VEC, VEC)
        ex_v[sl] = jnp.exp(lg_v[sl] - g)
        return 0
    lax.fori_loop(0, EP // VEC, estep, 0)

    plsc.subcore_barrier()

    def sstep(j, _):
        pltpu.sync_copy(ex_v.at[pl.ds(j * B, B)],
                        den_sh.at[dst2_v.at[j]], add=True)
        return 0
    lax.fori_loop(0, NBLK, sstep, 0)

    plsc.subcore_barrier()
    pltpu.sync_copy(den_sh.at[pl.ds(sid * NPT, NPT)],
                    den_h.at[cid, pl.ds(sid * NPT, NPT)])


@functools.partial(
    pl.kernel,
    out_type=jax.ShapeDtypeStruct((E,), jnp.float32),
    mesh=_mesh,
    compiler_params=_sc_params,
    scratch_types=(pltpu.VMEM((EP,), jnp.float32),
                   pltpu.VMEM((NBLK, B), jnp.int32),
                   pltpu.VMEM((NP,), jnp.float32)),
)
def _sc_alpha(lg_h, dst3_h, mt_h, al_h, lg_v, dst2_v, mt_v):
    wid = _wid()
    base = wid * EP
    pltpu.sync_copy(lg_h.at[pl.ds(base, EP)], lg_v)
    pltpu.sync_copy(dst3_h.at[wid], dst2_v)
    pltpu.sync_copy(mt_h, mt_v)

    def astep(j, _):
        for k in range(B // VEC):
            sl = pl.ds(j * B + k * VEC, VEC)
            d16 = dst2_v[j, pl.ds(k * VEC, VEC)]
            m16 = plsc.load_gather(mt_v, [d16])
            lg_v[sl] = jnp.exp(lg_v[sl] - m16)
        return 0
    lax.fori_loop(0, NBLK, astep, 0)
    pltpu.sync_copy(lg_v, al_h.at[pl.ds(base, EP)])


@functools.partial(
    pl.kernel,
    out_type=jax.ShapeDtypeStruct((NC, NP, D), jnp.float32),
    mesh=_mesh,
    compiler_params=_sc_params,
    scratch_types=(pltpu.VMEM((NBLK, B), jnp.int32),
                   pltpu.VMEM((B, D), jnp.float32),
                   pltpu.VMEM((B, D), jnp.float32),
                   pltpu.VMEM((B,), jnp.int32),
                   pltpu.VMEM((B,), jnp.int32),
                   pltpu.VMEM((B,), jnp.int32),
                   pltpu.VMEM((B,), jnp.int32),
                   pltpu.VMEM((B,), jnp.float32),
                   pltpu.VMEM((B,), jnp.float32),
                   pltpu.VMEM_SHARED((NP, D), jnp.float32),
                   pltpu.SemaphoreType.DMA,
                   pltpu.SemaphoreType.DMA,
                   pltpu.SemaphoreType.DMA,
                   pltpu.SemaphoreType.DMA,
                   pltpu.SemaphoreType.DMA,
                   pltpu.SemaphoreType.DMA),
)
def _sc_out(al_h, pk3_h, hw_h, out_h,
            pk2_v, rb0, rb1, sr0, sr1, dr0, dr1, ar0, ar1,
            out_sh, g0, g1, s0, s1, a0, a1):
    cid = lax.axis_index("c")
    sid = lax.axis_index("s")
    wid = _wid()
    base = wid * EP
    pltpu.sync_copy(pk3_h.at[wid], pk2_v)

    rbs = (rb0, rb1)
    srs = (sr0, sr1)
    drs = (dr0, dr1)
    ars = (ar0, ar1)
    gsem = (g0, g1)
    ssem = (s0, s1)
    asem = (a0, a1)

    def unpack(j, b):
        for k in range(B // VEC):
            sl = pl.ds(k * VEC, VEC)
            p16 = pk2_v[j, sl]
            srs[b][sl] = p16 & 16383
            drs[b][sl] = p16 >> 14

    def stage1(j, b):
        pltpu.make_async_copy(rbs[b], out_sh.at[drs[b]], ssem[b]).wait()
        unpack(j, b)
        pltpu.async_copy(al_h.at[pl.ds(base + j * B, B)], ars[b], asem[b])
        pltpu.async_copy(hw_h.at[srs[b]], rbs[b], gsem[b])

    def stage2(j, b):
        pltpu.make_async_copy(hw_h.at[srs[b]], rbs[b], gsem[b]).wait()
        pltpu.make_async_copy(al_h.at[pl.ds(base + j * B, B)], ars[b],
                              asem[b]).wait()

        def row16(t, _):
            a16 = ars[b][pl.ds(t * VEC, VEC)]
            for i in range(VEC):
                a = a16[i]
                r = t * VEC + i
                for c in range(8):
                    sl = pl.ds(c * VEC, VEC)
                    rbs[b][r, sl] = rbs[b][r, sl] * a
            return 0
        lax.fori_loop(0, B // VEC, row16, 0)
        pltpu.async_copy(rbs[b], out_sh.at[drs[b]], ssem[b], add=True)

    def zrow(r, _):
        for c in range(8):
            sl = pl.ds(c * VEC, VEC)
            rb0[r, sl] = jnp.zeros((VEC,), jnp.float32)
            rb1[r, sl] = jnp.zeros((VEC,), jnp.float32)
        return 0
    lax.fori_loop(0, B, zrow, 0)
    for k in range(NPT // B):
        pltpu.sync_copy(rb0, out_sh.at[pl.ds(sid * NPT + k * B, B)])
    plsc.subcore_barrier()

    unpack(0, 0)
    unpack(1, 1)
    pltpu.async_copy(rb0, out_sh.at[dr0], s0, add=True)
    pltpu.async_copy(rb1, out_sh.at[dr1], s1, add=True)
    stage1(0, 0)
    stage1(1, 1)

    def pair(jj, _):
        j0 = 2 * jj
        stage2(j0, 0)
        stage1(j0 + 2, 0)
        stage2(j0 + 1, 1)
        stage1(j0 + 3, 1)
        return 0
    lax.fori_loop(0, (NBLK - 3) // 2, pair, 0)
    stage2(122, 0)
    stage1(124, 0)
    stage2(123, 1)
    stage2(124, 0)
    pltpu.make_async_copy(rb0, out_sh.at[dr0], s0).wait()
    pltpu.make_async_copy(rb1, out_sh.at[dr1], s1).wait()

    plsc.subcore_barrier()
    pltpu.sync_copy(out_sh.at[pl.ds(sid * NPT, NPT)],
                    out_h.at[cid, pl.ds(sid * NPT, NPT)])


def kernel(x, edge_index, edge_attr, batch, fc_node_W, fc_node_b, fc_edge_W,
           fc_edge_b, bn_node_g, bn_node_b, bn_edge_g, bn_edge_b, gat_W,
           gat_a_src, gat_a_dst, gat_a_edge, gat_b):
    src = edge_index[0]
    dst = edge_index[1]
    dst3 = dst.reshape(NW, NBLK, B)
    pk3 = (src + dst * 16384).reshape(NW, NBLK, B)

    xtx, xcs = _stats(x, 1000)
    xmean = xcs[0] / N
    mean_v = xmean @ fc_node_W + fc_node_b
    exx = xtx / N
    var_v = jnp.sum((exx @ fc_node_W) * fc_node_W, axis=0) - (mean_v - fc_node_b) ** 2
    scale = bn_node_g / jnp.sqrt(var_v + 1e-5)
    M2 = fc_node_W * scale[None, :]
    c2 = (fc_node_b - mean_v) * scale + bn_node_b

    ete, ecs = _stats(edge_attr, 8000)
    emean = ecs[0] / E
    mean_e = emean @ fc_edge_W + fc_edge_b
    eee = ete / E
    var_e = jnp.sum((eee @ fc_edge_W) * fc_edge_W, axis=0) - (mean_e - fc_edge_b) ** 2
    scale_e = bn_edge_g / jnp.sqrt(var_e + 1e-5)
    A = jnp.zeros((DE, 8), jnp.float32).at[:, :L].set(gat_a_edge.T)
    M3 = (fc_edge_W * scale_e[None, :]) @ A
    c3 = ((fc_edge_b - mean_e) * scale_e + bn_edge_b) @ A
    ea_all = _affine(edge_attr, M3, c3, 8000)

    h = _affine(x, M2, c2, 1000)

    out_prev = None
    for l in range(L):
        a2 = jnp.zeros((D, 8), jnp.float32)
        a2 = a2.at[:, 0].set(gat_a_src[l]).at[:, 1].set(gat_a_dst[l])
        if l == 0:
            hw, sd = _layer_in(h, gat_W[l], a2)
        else:
            h, hw, sd = _layer_upd(h, out_prev[0, :N], out_prev[1, :N],
                                   gat_b[l - 1], gat_W[l], a2)
        s = sd[:, 0]
        d = sd[:, 1]
        eal = ea_all[:, l]

        logits, tmax = _sc_logits(s, d, src, dst, eal)
        den_p = _sc_den(logits, dst3, tmax)
        mt = _mt(den_p, tmax)
        alpha = _sc_alpha(logits, dst3, mt)
        out_prev = _sc_out(alpha, pk3, hw)

    h = _update(h, out_prev[0, :N], out_prev[1, :N], gat_b[L - 1])
    return h.reshape(BS, -1, D)
